# baseline (speedup 1.0000x reference)
import functools

import jax
import jax.numpy as jnp
from jax import lax
from jax.experimental import pallas as pl
from jax.experimental.pallas import tpu as pltpu
from jax.experimental.pallas import tpu_sc as plsc

E = 160000
D = 256
L = 16
NW = 32
EPW = E // NW
C = 16
NCH = -(-EPW // C)
NROW = NCH + 1
CPAD = NCH * C


def _dot_body(h_hbm, src_hbm, dst_hbm, out_hbm, src_v, dst_v, u0, v0, u1, v1,
              out_v, sem0, sem1):
    wid = lax.axis_index("s") * 2 + lax.axis_index("c")
    base = wid * EPW
    pltpu.sync_copy(src_hbm.at[wid], src_v)
    pltpu.sync_copy(dst_hbm.at[wid], dst_v)

    def start(j, us, vs, sem):
        pltpu.async_copy(h_hbm.at[src_v.at[j]], us, sem)
        pltpu.async_copy(h_hbm.at[dst_v.at[j]], vs, sem)

    def wait(us, vs, sem):
        pltpu.make_async_copy(h_hbm.at[src_v.at[0]], us, sem).wait()
        pltpu.make_async_copy(h_hbm.at[src_v.at[0]], vs, sem).wait()

    lane = lax.iota(jnp.int32, L)

    def compute_chunk(u_v, v_v, j):
        def group_body(g, _):
            def edge_body(i, gvec):
                e = g * L + i
                p = []
                for k in range(D // (2 * L)):
                    uw = plsc.bitcast(u_v[e, pl.ds(k * L, L)], jnp.bfloat16)
                    vw = plsc.bitcast(v_v[e, pl.ds(k * L, L)], jnp.bfloat16)
                    ua, ub = plsc.unpack(uw, format=plsc.PackFormat.INTERLEAVED)
                    va, vb = plsc.unpack(vw, format=plsc.PackFormat.INTERLEAVED)
                    p.append(ua * va)
                    p.append(ub * vb)
                while len(p) > 1:
                    p = [p[a] + p[a + 1] for a in range(0, len(p), 2)]
                return jnp.where(lane == i, jnp.sum(p[0]), gvec)

            gvec = lax.fori_loop(0, L, edge_body,
                                 jnp.zeros((L,), jnp.float32), unroll=4)
            out_v[pl.ds(j * C + g * L, L)] = gvec
            return 0

        lax.fori_loop(0, C // L, group_body, 0)

    start(0, u0, v0, sem0)
    start(1, u1, v1, sem1)

    def pair_body(i, _):
        j0 = 2 * i
        wait(u0, v0, sem0)
        compute_chunk(u0, v0, j0)
        start(j0 + 2, u0, v0, sem0)
        wait(u1, v1, sem1)
        compute_chunk(u1, v1, j0 + 1)
        start(j0 + 3, u1, v1, sem1)
        return 0

    lax.fori_loop(0, (NCH - 1) // 2, pair_body, 0)

    wait(u0, v0, sem0)
    compute_chunk(u0, v0, NCH - 1)
    wait(u1, v1, sem1)
    pltpu.sync_copy(out_v.at[pl.ds(0, EPW)], out_hbm.at[pl.ds(base, EPW)])


_dot_kernel = functools.partial(
    pl.kernel,
    out_type=jax.ShapeDtypeStruct((E,), jnp.float32),
    mesh=plsc.VectorSubcoreMesh(core_axis_name="c", subcore_axis_name="s"),
    compiler_params=pltpu.CompilerParams(needs_layout_passes=False),
    scratch_types=[
        pltpu.VMEM((NROW, C), jnp.int32),
        pltpu.VMEM((NROW, C), jnp.int32),
        pltpu.VMEM((C, D // 2), jnp.int32),
        pltpu.VMEM((C, D // 2), jnp.int32),
        pltpu.VMEM((C, D // 2), jnp.int32),
        pltpu.VMEM((C, D // 2), jnp.int32),
        pltpu.VMEM((CPAD,), jnp.float32),
        pltpu.SemaphoreType.DMA,
        pltpu.SemaphoreType.DMA,
    ],
)(_dot_body)


@jax.jit
def kernel(h, edge_index):
    pad = ((0, 0), (0, NROW * C - EPW))
    src = jnp.pad(edge_index[0].astype(jnp.int32).reshape(NW, EPW), pad)
    dst = jnp.pad(edge_index[1].astype(jnp.int32).reshape(NW, EPW), pad)
    hb = lax.bitcast_convert_type(
        h.astype(jnp.bfloat16).reshape(-1, D // 2, 2), jnp.int32)
    return _dot_kernel(hb,
                       src.reshape(NW, NROW, C), dst.reshape(NW, NROW, C))


# speedup vs baseline: 1.1779x; 1.1779x over previous
import functools

import jax
import jax.numpy as jnp
from jax import lax
from jax.experimental import pallas as pl
from jax.experimental.pallas import tpu as pltpu
from jax.experimental.pallas import tpu_sc as plsc

E = 160000
D = 256
W = D // 2
L = 16
NW = 32
EPW = E // NW
C = 16
NCH = -(-EPW // C)
NBUF = 4
NROW = NCH + NBUF - 1
CPAD = NCH * C


def _dot_body(h_hbm, src_hbm, dst_hbm, out_hbm, src_v, dst_v,
              u_bufs, v_bufs, out_v, sems):
    wid = lax.axis_index("s") * 2 + lax.axis_index("c")
    base = wid * EPW
    pltpu.sync_copy(src_hbm.at[wid], src_v)
    pltpu.sync_copy(dst_hbm.at[wid], dst_v)

    def start(j, b):
        pltpu.async_copy(h_hbm.at[src_v.at[j]], u_bufs[b], sems[b])
        pltpu.async_copy(h_hbm.at[dst_v.at[j]], v_bufs[b], sems[b])

    def wait(b):
        pltpu.make_async_copy(h_hbm.at[src_v.at[0]], u_bufs[b], sems[b]).wait()
        pltpu.make_async_copy(h_hbm.at[src_v.at[0]], v_bufs[b], sems[b]).wait()

    lane = lax.iota(jnp.int32, L)

    def compute_chunk(u_v, v_v, j):
        def edge_body(i, gvec):
            acc = [None] * 4
            for k in range(W // L):
                uw = plsc.bitcast(u_v[i, pl.ds(k * L, L)], jnp.bfloat16)
                vw = plsc.bitcast(v_v[i, pl.ds(k * L, L)], jnp.bfloat16)
                ua, ub = plsc.unpack(uw, format=plsc.PackFormat.INTERLEAVED)
                va, vb = plsc.unpack(vw, format=plsc.PackFormat.INTERLEAVED)
                j0, j1 = (2 * k) % 4, (2 * k + 1) % 4
                pa, pb = ua * va, ub * vb
                acc[j0] = pa if acc[j0] is None else acc[j0] + pa
                acc[j1] = pb if acc[j1] is None else acc[j1] + pb
            tot = (acc[0] + acc[1]) + (acc[2] + acc[3])
            return jnp.where(lane == i, jnp.sum(tot), gvec)

        gvec = lax.fori_loop(0, L, edge_body, jnp.zeros((L,), jnp.float32))
        out_v[pl.ds(j * C, L)] = gvec

    for b in range(NBUF):
        start(b, b)

    def ring_body(i, _):
        j = NBUF * i
        for b in range(NBUF):
            wait(b)
            compute_chunk(u_bufs[b], v_bufs[b], j + b)
            start(j + b + NBUF, b)
        return 0

    lax.fori_loop(0, (NCH - 1) // NBUF, ring_body, 0)

    wait(0)
    compute_chunk(u_bufs[0], v_bufs[0], NCH - 1)
    for b in range(1, NBUF):
        wait(b)
    pltpu.sync_copy(out_v.at[pl.ds(0, EPW)], out_hbm.at[pl.ds(base, EPW)])


def _body_wrapper(h_hbm, src_hbm, dst_hbm, out_hbm,
                  src_v, dst_v,
                  u0, u1, u2, u3, v0, v1, v2, v3, out_v,
                  s0, s1, s2, s3):
    _dot_body(h_hbm, src_hbm, dst_hbm, out_hbm, src_v, dst_v,
              [u0, u1, u2, u3], [v0, v1, v2, v3], out_v, [s0, s1, s2, s3])


_dot_kernel = functools.partial(
    pl.kernel,
    out_type=jax.ShapeDtypeStruct((E,), jnp.float32),
    mesh=plsc.VectorSubcoreMesh(core_axis_name="c", subcore_axis_name="s"),
    compiler_params=pltpu.CompilerParams(needs_layout_passes=False),
    scratch_types=(
        [pltpu.VMEM((NROW, C), jnp.int32)] * 2
        + [pltpu.VMEM((C, W), jnp.int32)] * (2 * NBUF)
        + [pltpu.VMEM((CPAD,), jnp.float32)]
        + [pltpu.SemaphoreType.DMA] * NBUF
    ),
)(_body_wrapper)


@jax.jit
def kernel(h, edge_index):
    pad = ((0, 0), (0, NROW * C - EPW))
    src = jnp.pad(edge_index[0].astype(jnp.int32).reshape(NW, EPW), pad)
    dst = jnp.pad(edge_index[1].astype(jnp.int32).reshape(NW, EPW), pad)
    hb = lax.bitcast_convert_type(
        h.astype(jnp.bfloat16).reshape(-1, W, 2), jnp.int32)
    return _dot_kernel(hb,
                       src.reshape(NW, NROW, C), dst.reshape(NW, NROW, C))


# speedup vs baseline: 1.3920x; 1.1817x over previous
import functools

import jax
import jax.numpy as jnp
from jax import lax
from jax.experimental import pallas as pl
from jax.experimental.pallas import tpu as pltpu
from jax.experimental.pallas import tpu_sc as plsc

E = 160000
D = 256
L = 16
NW = 32
EPW = E // NW
C = 16
NCH = -(-EPW // C)
NROW = NCH + 1
CPAD = NCH * C


def _dot_body(h_hbm, src_hbm, dst_hbm, out_hbm, src_v, dst_v, u0, v0, u1, v1,
              out_v, sem0, sem1):
    wid = lax.axis_index("s") * 2 + lax.axis_index("c")
    base = wid * EPW
    pltpu.sync_copy(src_hbm.at[wid], src_v)
    pltpu.sync_copy(dst_hbm.at[wid], dst_v)

    def start(j, us, vs, sem):
        pltpu.async_copy(h_hbm.at[src_v.at[j]], us, sem)
        pltpu.async_copy(h_hbm.at[dst_v.at[j]], vs, sem)

    def wait(us, vs, sem):
        pltpu.make_async_copy(h_hbm.at[src_v.at[0]], us, sem).wait()
        pltpu.make_async_copy(h_hbm.at[src_v.at[0]], vs, sem).wait()

    lane = lax.iota(jnp.int32, L)

    def compute_chunk(u_v, v_v, j):
        def edge_body(i, gvec):
            p = [u_v[i, pl.ds(k * L, L)] * v_v[i, pl.ds(k * L, L)]
                 for k in range(D // L)]
            while len(p) > 1:
                p = [p[a] + p[a + 1] for a in range(0, len(p), 2)]
            return jnp.where(lane == i, jnp.sum(p[0]), gvec)

        gvec = lax.fori_loop(0, L, edge_body, jnp.zeros((L,), jnp.float32))
        out_v[pl.ds(j * C, L)] = gvec

    start(0, u0, v0, sem0)
    start(1, u1, v1, sem1)

    def pair_body(i, _):
        j0 = 2 * i
        wait(u0, v0, sem0)
        compute_chunk(u0, v0, j0)
        start(j0 + 2, u0, v0, sem0)
        wait(u1, v1, sem1)
        compute_chunk(u1, v1, j0 + 1)
        start(j0 + 3, u1, v1, sem1)
        return 0

    lax.fori_loop(0, (NCH - 1) // 2, pair_body, 0)

    wait(u0, v0, sem0)
    compute_chunk(u0, v0, NCH - 1)
    wait(u1, v1, sem1)
    pltpu.sync_copy(out_v.at[pl.ds(0, EPW)], out_hbm.at[pl.ds(base, EPW)])


_dot_kernel = functools.partial(
    pl.kernel,
    out_type=jax.ShapeDtypeStruct((E,), jnp.float32),
    mesh=plsc.VectorSubcoreMesh(core_axis_name="c", subcore_axis_name="s"),
    compiler_params=pltpu.CompilerParams(needs_layout_passes=False),
    scratch_types=[
        pltpu.VMEM((NROW, C), jnp.int32),
        pltpu.VMEM((NROW, C), jnp.int32),
        pltpu.VMEM((C, D), jnp.float32),
        pltpu.VMEM((C, D), jnp.float32),
        pltpu.VMEM((C, D), jnp.float32),
        pltpu.VMEM((C, D), jnp.float32),
        pltpu.VMEM((CPAD,), jnp.float32),
        pltpu.SemaphoreType.DMA,
        pltpu.SemaphoreType.DMA,
    ],
)(_dot_body)


@jax.jit
def kernel(h, edge_index):
    pad = ((0, 0), (0, NROW * C - EPW))
    src = jnp.pad(edge_index[0].astype(jnp.int32).reshape(NW, EPW), pad)
    dst = jnp.pad(edge_index[1].astype(jnp.int32).reshape(NW, EPW), pad)
    return _dot_kernel(h, src.reshape(NW, NROW, C), dst.reshape(NW, NROW, C))
